# Initial kernel scaffold; baseline (speedup 1.0000x reference)
import functools

import jax
import jax.numpy as jnp
from jax import lax
from jax.experimental import pallas as pl
from jax.experimental.pallas import tpu as pltpu
from jax.experimental.pallas import tpu_sc as plsc

N = 10000
FCH = 128
NTILES = 16
BATCH = 128
RPAD = 10240
RPT = RPAD // NTILES
DUMMY = RPAD - 1
RB = 1000
GRID = N // RB

_mesh = plsc.VectorSubcoreMesh(core_axis_name="c", subcore_axis_name="s")


def _zero2d(ref):
    z = jnp.zeros((16,), jnp.float32)

    def body(i, carry):
        ref[i // 8, pl.ds((i % 8) * 16, 16)] = z
        return carry

    lax.fori_loop(0, 128 * 8, body, 0)


def _fill1d(ref, n16, value):
    v = jnp.full((16,), value, jnp.float32)

    def body(i, carry):
        ref[pl.ds(i * 16, 16)] = v
        return carry

    lax.fori_loop(0, n16, body, 0)


def _writeout(t, acc, out_hbm):
    last = (NTILES - 1) * RPT
    tail = N - last

    @pl.when(t < NTILES - 1)
    def _():
        pltpu.sync_copy(acc.at[pl.ds(t * RPT, RPT)], out_hbm.at[pl.ds(t * RPT, RPT)])

    @pl.when(t == NTILES - 1)
    def _():
        pltpu.sync_copy(acc.at[pl.ds(last, tail)], out_hbm.at[pl.ds(last, tail)])


def _make_segsum(nchunks, with_deg, nb):
    cpc = nchunks // 2
    out_type = [jax.ShapeDtypeStruct((N, FCH), jnp.float32) for _ in range(nchunks)]
    if with_deg:
        out_type.append(jax.ShapeDtypeStruct((N,), jnp.float32))
    scratch = [
        pltpu.VMEM_SHARED((RPAD, FCH), jnp.float32),
        pltpu.VMEM((nb, BATCH), jnp.int32),
        pltpu.VMEM((nb, BATCH), jnp.int32),
        pltpu.VMEM((BATCH, FCH), jnp.float32),
        pltpu.VMEM((128, 128), jnp.float32),
        pltpu.SemaphoreType.DMA,
    ]
    if with_deg:
        scratch += [
            pltpu.VMEM_SHARED((RPAD,), jnp.float32),
            pltpu.VMEM((RPT,), jnp.float32),
            pltpu.VMEM((BATCH,), jnp.float32),
        ]

    @functools.partial(pl.kernel, out_type=out_type, mesh=_mesh,
                       scratch_types=scratch)
    def seg(*args):
        xs = args[:nchunks]
        srcw, dstw = args[nchunks], args[nchunks + 1]
        outs = args[nchunks + 2:2 * nchunks + 2]
        rest = args[2 * nchunks + 2:]
        if with_deg:
            deg_out = rest[0]
            acc, srcv, dstv, rows, zrows, sem, cnt, zcnt, ones = rest[1:]
        else:
            acc, srcv, dstv, rows, zrows, sem = rest

        c = lax.axis_index("c")
        t = lax.axis_index("s")

        pltpu.sync_copy(srcw.at[t], srcv)
        pltpu.sync_copy(dstw.at[t], dstv)
        _zero2d(zrows)
        if with_deg:
            _fill1d(zcnt, RPT // 16, 0.0)
            _fill1d(ones, BATCH // 16, 1.0)

        def run(x_hbm, do_deg):
            def body(b, carry):
                pltpu.async_copy(x_hbm.at[srcv.at[b]], rows, sem).wait()
                pltpu.sync_copy(rows, acc.at[dstv.at[b]], add=True)
                if do_deg:
                    pltpu.sync_copy(ones, cnt.at[dstv.at[b]], add=True)
                return carry
            lax.fori_loop(0, nb, body, 0)

        def phase(x_hbm, out_hbm, do_deg):
            for j in range(RPT // 128):
                pltpu.sync_copy(zrows, acc.at[pl.ds(t * RPT + j * 128, 128)])
            if do_deg:
                pltpu.sync_copy(zcnt, cnt.at[pl.ds(t * RPT, RPT)])
            plsc.subcore_barrier()
            run(x_hbm, do_deg)
            plsc.subcore_barrier()
            _writeout(t, acc, out_hbm)
            if do_deg:
                _writeout(t, cnt, deg_out)
            plsc.subcore_barrier()

        for p in range(cpc):
            @pl.when(c == 0)
            def _(p=p):
                phase(xs[p], outs[p], with_deg and p == 0)

            @pl.when(c == 1)
            def _(p=p):
                phase(xs[cpc + p], outs[cpc + p], False)

    return seg


_SELU_SCALE = 1.0507009873554805
_SELU_ALPHA = 1.6732632423543772


def _thresh(v):
    v = jnp.where(v > 1.0, v - 1.0, v)
    v = jnp.where(v <= -1.0, v + 1.0, v)
    return jnp.where((v >= -1.0) & (v <= 1.0), jnp.zeros_like(v), v)


def _t0_body(s0, s1, d, h0, h1):
    recip = 1.0 / jnp.maximum(d[...], 1.0)
    h0[...] = _thresh(s0[...] * recip)
    h1[...] = _thresh(s1[...] * recip)


def _t1_body(a0, a1, d, h0, h1, wl, wr, b1, o0, o1, o2, o3):
    recip = 1.0 / jnp.maximum(d[...], 1.0)
    wl_ = wl[...]
    wr_ = wr[...]
    acc = jnp.dot(a0[...] * recip, wl_[:FCH], preferred_element_type=jnp.float32)
    acc += jnp.dot(a1[...] * recip, wl_[FCH:], preferred_element_type=jnp.float32)
    acc += jnp.dot(h0[...], wr_[:FCH], preferred_element_type=jnp.float32)
    acc += jnp.dot(h1[...], wr_[FCH:], preferred_element_type=jnp.float32)
    acc += b1[...]
    act = _SELU_SCALE * jnp.where(acc > 0.0, acc,
                                  _SELU_ALPHA * (jnp.exp(acc) - 1.0))
    o0[...] = act[:, 0 * FCH:1 * FCH]
    o1[...] = act[:, 1 * FCH:2 * FCH]
    o2[...] = act[:, 2 * FCH:3 * FCH]
    o3[...] = act[:, 3 * FCH:4 * FCH]


def _t2_body(u0, u1, u2, u3, d, g0, g1, g2, g3, wl, wr, b2, out):
    recip = 1.0 / jnp.maximum(d[...], 1.0)
    wl_ = wl[...]
    wr_ = wr[...]
    acc = jnp.zeros((RB, 64), jnp.float32) + b2[...]
    for j, (u, g) in enumerate(((u0, g0), (u1, g1), (u2, g2), (u3, g3))):
        acc += jnp.dot(u[...] * recip, wl_[j * FCH:(j + 1) * FCH],
                       preferred_element_type=jnp.float32)
        acc += jnp.dot(g[...], wr_[j * FCH:(j + 1) * FCH],
                       preferred_element_type=jnp.float32)
    m = jnp.max(acc, axis=1, keepdims=True)
    e = jnp.exp(acc - m)
    out[...] = e / jnp.sum(e, axis=1, keepdims=True)


def _row_spec(w):
    return pl.BlockSpec((RB, w), lambda i: (i, 0))


def _full_spec(r, c):
    return pl.BlockSpec((r, c), lambda i: (0, 0))


def _tc_t0(s0, s1, deg2):
    return pl.pallas_call(
        _t0_body,
        grid=(GRID,),
        in_specs=[_row_spec(FCH), _row_spec(FCH), _row_spec(1)],
        out_specs=[_row_spec(FCH), _row_spec(FCH)],
        out_shape=[jax.ShapeDtypeStruct((N, FCH), jnp.float32)] * 2,
    )(s0, s1, deg2)


def _tc_t1(a0, a1, deg2, h0, h1, wl, wr, b1):
    return pl.pallas_call(
        _t1_body,
        grid=(GRID,),
        in_specs=[_row_spec(FCH), _row_spec(FCH), _row_spec(1),
                  _row_spec(FCH), _row_spec(FCH),
                  _full_spec(256, 512), _full_spec(256, 512), _full_spec(1, 512)],
        out_specs=[_row_spec(FCH)] * 4,
        out_shape=[jax.ShapeDtypeStruct((N, FCH), jnp.float32)] * 4,
    )(a0, a1, deg2, h0, h1, wl, wr, b1)


def _tc_t2(us, deg2, gs, wl, wr, b2):
    return pl.pallas_call(
        _t2_body,
        grid=(GRID,),
        in_specs=[_row_spec(FCH)] * 4 + [_row_spec(1)] + [_row_spec(FCH)] * 4
                 + [_full_spec(512, 64), _full_spec(512, 64), _full_spec(1, 64)],
        out_specs=_row_spec(64),
        out_shape=jax.ShapeDtypeStruct((N, 64), jnp.float32),
    )(*us, deg2, *gs, wl, wr, b2)


def kernel(x, edge_index, W1l, b1, W1r, W2l, b2, W2r):
    src = edge_index[0].astype(jnp.int32)
    dst = edge_index[1].astype(jnp.int32)
    e = src.shape[0]
    grain = NTILES * BATCH * 2
    ep = -(-e // grain) * grain
    nb = ep // (NTILES * BATCH)
    srcw = jnp.concatenate(
        [src, jnp.zeros((ep - e,), jnp.int32)]).reshape(NTILES, nb, BATCH)
    dstw = jnp.concatenate(
        [dst, jnp.full((ep - e,), DUMMY, jnp.int32)]).reshape(NTILES, nb, BATCH)

    x0 = x[:, :FCH]
    x1 = x[:, FCH:]

    seg2d = _make_segsum(2, True, nb)
    seg2 = _make_segsum(2, False, nb)
    seg4 = _make_segsum(4, False, nb)

    s0, s1, deg = seg2d(x0, x1, srcw, dstw)
    deg2 = deg[:, None]

    h0, h1 = _tc_t0(s0, s1, deg2)

    a0, a1 = seg2(h0, h1, srcw, dstw)

    g0, g1, g2, g3 = _tc_t1(a0, a1, deg2, h0, h1,
                            W1l.T, W1r.T, b1[None, :])

    u0, u1, u2, u3 = seg4(g0, g1, g2, g3, srcw, dstw)

    return _tc_t2((u0, u1, u2, u3), deg2, (g0, g1, g2, g3),
                  W2l.T, W2r.T, b2[None, :])


# speedup vs baseline: 2.7033x; 2.7033x over previous
import functools

import jax
import jax.numpy as jnp
from jax import lax
from jax.experimental import pallas as pl
from jax.experimental.pallas import tpu as pltpu
from jax.experimental.pallas import tpu_sc as plsc

N = 10000
FCH = 128
NTILES = 16
BATCH = 128
RPAD = 10240
RPT = RPAD // NTILES
DUMMY = RPAD - 1
RB = 1024
GRID = RPAD // RB

_mesh = plsc.VectorSubcoreMesh(core_axis_name="c", subcore_axis_name="s")


def _zero2d(ref):
    z = jnp.zeros((16,), jnp.float32)

    def body(i, carry):
        ref[i // 8, pl.ds((i % 8) * 16, 16)] = z
        return carry

    lax.fori_loop(0, 128 * 8, body, 0)


def _fill1d(ref, n16, value):
    v = jnp.full((16,), value, jnp.float32)

    def body(i, carry):
        ref[pl.ds(i * 16, 16)] = v
        return carry

    lax.fori_loop(0, n16, body, 0)


def _writeout(t, acc, out_hbm):
    pltpu.sync_copy(acc.at[pl.ds(t * RPT, RPT)], out_hbm.at[pl.ds(t * RPT, RPT)])


def _make_segsum(nchunks, with_deg, nb):
    cpc = nchunks // 2
    out_type = [jax.ShapeDtypeStruct((RPAD, FCH), jnp.float32) for _ in range(nchunks)]
    if with_deg:
        out_type.append(jax.ShapeDtypeStruct((RPAD,), jnp.float32))
    scratch = [
        pltpu.VMEM_SHARED((RPAD, FCH), jnp.float32),
        pltpu.VMEM((nb, BATCH), jnp.int32),
        pltpu.VMEM((nb, BATCH), jnp.int32),
        pltpu.VMEM((BATCH, FCH), jnp.float32),
        pltpu.SemaphoreType.DMA,
    ]
    if with_deg:
        scratch += [
            pltpu.VMEM_SHARED((RPAD,), jnp.float32),
            pltpu.VMEM((RPT,), jnp.float32),
            pltpu.VMEM((BATCH,), jnp.float32),
        ]

    @functools.partial(pl.kernel, out_type=out_type, mesh=_mesh,
                       scratch_types=scratch)
    def seg(*args):
        xs = args[:nchunks]
        srcw, dstw = args[nchunks], args[nchunks + 1]
        outs = args[nchunks + 2:2 * nchunks + 2]
        rest = args[2 * nchunks + 2:]
        if with_deg:
            deg_out = rest[0]
            acc, srcv, dstv, rows, sem, cnt, zcnt, ones = rest[1:]
        else:
            acc, srcv, dstv, rows, sem = rest

        c = lax.axis_index("c")
        t = lax.axis_index("s")

        pltpu.sync_copy(srcw.at[t], srcv)
        pltpu.sync_copy(dstw.at[t], dstv)
        if with_deg:
            _fill1d(zcnt, RPT // 16, 0.0)
            _fill1d(ones, BATCH // 16, 1.0)

        def run(x_hbm, do_deg):
            def body(b, carry):
                pltpu.async_copy(x_hbm.at[srcv.at[b]], rows, sem).wait()
                pltpu.sync_copy(rows, acc.at[dstv.at[b]], add=True)
                if do_deg:
                    pltpu.sync_copy(ones, cnt.at[dstv.at[b]], add=True)
                return carry
            lax.fori_loop(0, nb, body, 0)

        def phase(x_hbm, out_hbm, do_deg):
            _zero2d(rows)
            for j in range(RPT // 128):
                pltpu.sync_copy(rows, acc.at[pl.ds(t * RPT + j * 128, 128)])
            if do_deg:
                pltpu.sync_copy(zcnt, cnt.at[pl.ds(t * RPT, RPT)])
            plsc.subcore_barrier()
            run(x_hbm, do_deg)
            plsc.subcore_barrier()
            _writeout(t, acc, out_hbm)
            if do_deg:
                _writeout(t, cnt, deg_out)
            plsc.subcore_barrier()

        for p in range(cpc):
            @pl.when(c == 0)
            def _(p=p):
                phase(xs[p], outs[p], with_deg and p == 0)

            @pl.when(c == 1)
            def _(p=p):
                phase(xs[cpc + p], outs[cpc + p], False)

    return seg


_SELU_SCALE = 1.0507009873554805
_SELU_ALPHA = 1.6732632423543772


def _thresh(v):
    v = jnp.where(v > 1.0, v - 1.0, v)
    v = jnp.where(v <= -1.0, v + 1.0, v)
    return jnp.where((v >= -1.0) & (v <= 1.0), jnp.zeros_like(v), v)


def _t0_body(s0, s1, d, h0, h1):
    recip = 1.0 / jnp.maximum(d[...], 1.0)
    h0[...] = _thresh(s0[...] * recip)
    h1[...] = _thresh(s1[...] * recip)


def _t1_body(a0, a1, d, h0, h1, wl, wr, b1, o0, o1, o2, o3):
    recip = 1.0 / jnp.maximum(d[...], 1.0)
    wl_ = wl[...]
    wr_ = wr[...]
    acc = jnp.dot(a0[...] * recip, wl_[:FCH], preferred_element_type=jnp.float32)
    acc += jnp.dot(a1[...] * recip, wl_[FCH:], preferred_element_type=jnp.float32)
    acc += jnp.dot(h0[...], wr_[:FCH], preferred_element_type=jnp.float32)
    acc += jnp.dot(h1[...], wr_[FCH:], preferred_element_type=jnp.float32)
    acc += b1[...]
    act = _SELU_SCALE * jnp.where(acc > 0.0, acc,
                                  _SELU_ALPHA * (jnp.exp(acc) - 1.0))
    o0[...] = act[:, 0 * FCH:1 * FCH]
    o1[...] = act[:, 1 * FCH:2 * FCH]
    o2[...] = act[:, 2 * FCH:3 * FCH]
    o3[...] = act[:, 3 * FCH:4 * FCH]


def _t2_body(u0, u1, u2, u3, d, g0, g1, g2, g3, wl, wr, b2, out):
    recip = 1.0 / jnp.maximum(d[...], 1.0)
    wl_ = wl[...]
    wr_ = wr[...]
    acc = jnp.zeros((RB, 64), jnp.float32) + b2[...]
    for j, (u, g) in enumerate(((u0, g0), (u1, g1), (u2, g2), (u3, g3))):
        acc += jnp.dot(u[...] * recip, wl_[j * FCH:(j + 1) * FCH],
                       preferred_element_type=jnp.float32)
        acc += jnp.dot(g[...], wr_[j * FCH:(j + 1) * FCH],
                       preferred_element_type=jnp.float32)
    m = jnp.max(acc, axis=1, keepdims=True)
    e = jnp.exp(acc - m)
    out[...] = e / jnp.sum(e, axis=1, keepdims=True)


def _row_spec(w):
    return pl.BlockSpec((RB, w), lambda i: (i, 0))


def _full_spec(r, c):
    return pl.BlockSpec((r, c), lambda i: (0, 0))


def _tc_t0(s0, s1, deg2):
    return pl.pallas_call(
        _t0_body,
        grid=(GRID,),
        in_specs=[_row_spec(FCH), _row_spec(FCH), _row_spec(1)],
        out_specs=[_row_spec(FCH), _row_spec(FCH)],
        out_shape=[jax.ShapeDtypeStruct((RPAD, FCH), jnp.float32)] * 2,
    )(s0, s1, deg2)


def _tc_t1(a0, a1, deg2, h0, h1, wl, wr, b1):
    return pl.pallas_call(
        _t1_body,
        grid=(GRID,),
        in_specs=[_row_spec(FCH), _row_spec(FCH), _row_spec(1),
                  _row_spec(FCH), _row_spec(FCH),
                  _full_spec(256, 512), _full_spec(256, 512), _full_spec(1, 512)],
        out_specs=[_row_spec(FCH)] * 4,
        out_shape=[jax.ShapeDtypeStruct((RPAD, FCH), jnp.float32)] * 4,
    )(a0, a1, deg2, h0, h1, wl, wr, b1)


def _tc_t2(us, deg2, gs, wl, wr, b2):
    return pl.pallas_call(
        _t2_body,
        grid=(GRID,),
        in_specs=[_row_spec(FCH)] * 4 + [_row_spec(1)] + [_row_spec(FCH)] * 4
                 + [_full_spec(512, 64), _full_spec(512, 64), _full_spec(1, 64)],
        out_specs=_row_spec(64),
        out_shape=jax.ShapeDtypeStruct((RPAD, 64), jnp.float32),
    )(*us, deg2, *gs, wl, wr, b2)


def kernel(x, edge_index, W1l, b1, W1r, W2l, b2, W2r):
    src = edge_index[0].astype(jnp.int32)
    dst = edge_index[1].astype(jnp.int32)
    e = src.shape[0]
    grain = NTILES * BATCH * 2
    ep = -(-e // grain) * grain
    nb = ep // (NTILES * BATCH)
    srcw = jnp.concatenate(
        [src, jnp.zeros((ep - e,), jnp.int32)]).reshape(NTILES, nb, BATCH)
    dstw = jnp.concatenate(
        [dst, jnp.full((ep - e,), DUMMY, jnp.int32)]).reshape(NTILES, nb, BATCH)

    x0 = x[:, :FCH]
    x1 = x[:, FCH:]

    seg2d = _make_segsum(2, True, nb)
    seg2 = _make_segsum(2, False, nb)
    seg4 = _make_segsum(4, False, nb)

    s0, s1, deg = seg2d(x0, x1, srcw, dstw)
    deg2 = deg[:, None]

    h0, h1 = _tc_t0(s0, s1, deg2)

    a0, a1 = seg2(h0, h1, srcw, dstw)

    g0, g1, g2, g3 = _tc_t1(a0, a1, deg2, h0, h1,
                            W1l.T, W1r.T, b1[None, :])

    u0, u1, u2, u3 = seg4(g0, g1, g2, g3, srcw, dstw)

    out = _tc_t2((u0, u1, u2, u3), deg2, (g0, g1, g2, g3),
                 W2l.T, W2r.T, b2[None, :])
    return out[:N]
